# baseline (speedup 1.0000x reference)
import jax
import jax.numpy as jnp
from jax import lax
from jax.experimental import pallas as pl
from jax.experimental.pallas import tpu as pltpu
from jax.experimental.pallas import tpu_sc as plsc

N_POINTS = 1048576
PLANE = 512
FDIM = 64
OUT_DIM = 3 * FDIM
ROWS_PER_PLANE = PLANE * PLANE

NC = 2
NS = 16
NW = NC * NS
PPW = N_POINTS // NW
C = 64
LANES = 16
NG = C // LANES
NCHUNK = PPW // C
WVLEN = 3 * NG * 4 * LANES


def _compute_idx(xv, idxv, wv):
    @plsc.parallel_loop(0, NG)
    def vec_body(v):
        sl = pl.ds(v * LANES, LANES)
        lo, w1, ow, sh0, sh1 = [], [], [], [], []
        for d in range(3):
            c = xv[d, sl]
            gc = c * 2.0 - 1.0
            t = (gc + 1.0) * 0.5 * float(PLANE - 1)
            t = jnp.clip(t, 0.0, float(PLANE - 1))
            i0 = t.astype(jnp.int32)
            w = t - i0.astype(jnp.float32)
            i1 = jnp.minimum(i0 + 1, PLANE - 1)
            lo.append(i0)
            w1.append(w)
            ow.append(1.0 - w)
            sh0.append(i0 * PLANE)
            sh1.append(i1 * PLANE)
        for i, (dw, dh) in enumerate(((1, 2), (0, 2), (0, 1))):
            idxv[i, sl] = sh0[dh] + lo[dw] + (i * ROWS_PER_PLANE)
            idxv[i, pl.ds(C + v * LANES, LANES)] = (
                sh1[dh] + lo[dw] + (i * ROWS_PER_PLANE))
            wbase = ((i * NG + v) * 4) * LANES
            wv[pl.ds(wbase + 0 * LANES, LANES)] = ow[dw] * ow[dh]
            wv[pl.ds(wbase + 1 * LANES, LANES)] = w1[dw] * ow[dh]
            wv[pl.ds(wbase + 2 * LANES, LANES)] = ow[dw] * w1[dh]
            wv[pl.ds(wbase + 3 * LANES, LANES)] = w1[dw] * w1[dh]


def _blend(rows, wv, outv):
    for i in range(3):

        @plsc.parallel_loop(0, NG)
        def blend_group(v, i=i):
            wbase = ((i * NG + v) * 4) * LANES
            w00v = wv[pl.ds(wbase + 0 * LANES, LANES)]
            w01v = wv[pl.ds(wbase + 1 * LANES, LANES)]
            w10v = wv[pl.ds(wbase + 2 * LANES, LANES)]
            w11v = wv[pl.ds(wbase + 3 * LANES, LANES)]
            for j in range(LANES):
                p = v * LANES + j
                w00 = w00v[j]
                w01 = w01v[j]
                w10 = w10v[j]
                w11 = w11v[j]
                for f in range(FDIM // LANES):
                    sl0 = pl.ds(f * LANES, LANES)
                    sl1 = pl.ds(FDIM + f * LANES, LANES)
                    acc = (rows[i, p, sl0] * w00
                           + rows[i, p, sl1] * w01
                           + rows[i, C + p, sl0] * w10
                           + rows[i, C + p, sl1] * w11)
                    outv[p, pl.ds(i * FDIM + f * LANES, LANES)] = acc


def _sc_body(x_hbm, tab_hbm, out_hbm, xv, idxv, wv, rows, outv, semg, semo):
    cid = lax.axis_index("c")
    sid = lax.axis_index("s")
    wid = sid * NC + cid
    base = wid * PPW

    def stage(g, b):
        pltpu.sync_copy(x_hbm.at[:, pl.ds(base + g * C, C)], xv.at[b])
        _compute_idx(xv.at[b], idxv.at[b], wv.at[b])
        for i in range(3):
            pltpu.async_copy(
                tab_hbm.at[idxv.at[b, i]], rows.at[b, i], semg.at[b])

    def finish(g, b, first):
        for i in range(3):
            pltpu.make_async_copy(
                tab_hbm.at[idxv.at[b, i]], rows.at[b, i], semg.at[b]).wait()

        @pl.when(jnp.logical_not(first))
        def _():
            pltpu.make_async_copy(
                outv.at[b], out_hbm.at[pl.ds(base, C)], semo.at[b]).wait()

        _blend(rows.at[b], wv.at[b], outv.at[b])
        pltpu.async_copy(
            outv.at[b], out_hbm.at[pl.ds(base + g * C, C)], semo.at[b])

    stage(0, 0)

    def chunk_body(g, carry):
        b = lax.rem(g, 2)

        @pl.when(g + 1 < NCHUNK)
        def _():
            stage(g + 1, 1 - b)

        finish(g, b, g < 2)
        return carry

    lax.fori_loop(0, NCHUNK, chunk_body, 0)
    for b in range(2):
        pltpu.make_async_copy(
            outv.at[b], out_hbm.at[pl.ds(base, C)], semo.at[b]).wait()


def _run(x, fm):
    xT = x.T
    tab = fm.reshape(3 * ROWS_PER_PLANE, FDIM)
    tab2 = jnp.concatenate([tab, jnp.roll(tab, -1, axis=0)], axis=1)
    mesh = plsc.VectorSubcoreMesh(core_axis_name="c", subcore_axis_name="s")
    kfn = pl.kernel(
        _sc_body,
        out_type=jax.ShapeDtypeStruct((N_POINTS, OUT_DIM), jnp.float32),
        mesh=mesh,
        scratch_types=[
            pltpu.VMEM((2, 3, C), jnp.float32),
            pltpu.VMEM((2, 3, 2 * C), jnp.int32),
            pltpu.VMEM((2, WVLEN), jnp.float32),
            pltpu.VMEM((2, 3, 2 * C, 2 * FDIM), jnp.float32),
            pltpu.VMEM((2, C, OUT_DIM), jnp.float32),
            pltpu.SemaphoreType.DMA((2,)),
            pltpu.SemaphoreType.DMA((2,)),
        ],
        compiler_params=pltpu.CompilerParams(use_tc_tiling_on_sc=False),
    )
    return kfn(xT, tab2)


def kernel(x, level, fm):
    del level
    return _run(x, fm)


# speedup vs baseline: 1.1216x; 1.1216x over previous
import jax
import jax.numpy as jnp
from jax import lax
from jax.experimental import pallas as pl
from jax.experimental.pallas import tpu as pltpu
from jax.experimental.pallas import tpu_sc as plsc

N_POINTS = 1048576
PLANE = 512
FDIM = 64
OUT_DIM = 3 * FDIM
ROWS_PER_PLANE = PLANE * PLANE

NC = 2
NS = 16
NW = NC * NS
PPW = N_POINTS // NW
C = 128
LANES = 16
NCHUNK = PPW // C


def _sc_body(x_hbm, tab_hbm, out_hbm, xv, idxv, wv, rows, outv, sems, semo):
    cid = lax.axis_index("c")
    sid = lax.axis_index("s")
    wid = sid * NC + cid
    base = wid * PPW

    def chunk_body(g, carry):
        start = base + g * C
        pltpu.sync_copy(x_hbm.at[:, pl.ds(start, C)], xv)

        @plsc.parallel_loop(0, C // LANES)
        def vec_body(v):
            off = v * LANES
            sl = pl.ds(off, LANES)
            lo, hi, w1, ow, sh0, sh1 = [], [], [], [], [], []
            for d in range(3):
                c = xv[d, sl]
                gc = c * 2.0 - 1.0
                t = (gc + 1.0) * 0.5 * float(PLANE - 1)
                t = jnp.clip(t, 0.0, float(PLANE - 1))
                i0 = t.astype(jnp.int32)
                w = t - i0.astype(jnp.float32)
                i1 = jnp.minimum(i0 + 1, PLANE - 1)
                lo.append(i0)
                hi.append(i1)
                w1.append(w)
                ow.append(1.0 - w)
                sh0.append(i0 * PLANE)
                sh1.append(i1 * PLANE)
            for i, (dw, dh) in enumerate(((1, 2), (0, 2), (0, 1))):
                r0 = sh0[dh] + (i * ROWS_PER_PLANE)
                r1 = sh1[dh] + (i * ROWS_PER_PLANE)
                idxv[4 * i + 0, sl] = r0 + lo[dw]
                idxv[4 * i + 1, sl] = r0 + hi[dw]
                idxv[4 * i + 2, sl] = r1 + lo[dw]
                idxv[4 * i + 3, sl] = r1 + hi[dw]
                wbase = ((i * (C // LANES) + v) * 4) * LANES
                wv[pl.ds(wbase + 0 * LANES, LANES)] = ow[dw] * ow[dh]
                wv[pl.ds(wbase + 1 * LANES, LANES)] = w1[dw] * ow[dh]
                wv[pl.ds(wbase + 2 * LANES, LANES)] = ow[dw] * w1[dh]
                wv[pl.ds(wbase + 3 * LANES, LANES)] = w1[dw] * w1[dh]

        descs = []
        for i in range(3):
            descs.append([
                pltpu.async_copy(
                    tab_hbm.at[idxv.at[4 * i + cc]], rows.at[4 * i + cc],
                    sems.at[i])
                for cc in range(4)
            ])

        @pl.when(g > 0)
        def _():
            pltpu.make_async_copy(
                outv, out_hbm.at[pl.ds(base + (g - 1) * C, C)], semo).wait()

        for i in range(3):
            for d in descs[i]:
                d.wait()

            @plsc.parallel_loop(0, C // LANES)
            def blend_group(v, i=i):
                wbase = ((i * (C // LANES) + v) * 4) * LANES
                w00v = wv[pl.ds(wbase + 0 * LANES, LANES)]
                w01v = wv[pl.ds(wbase + 1 * LANES, LANES)]
                w10v = wv[pl.ds(wbase + 2 * LANES, LANES)]
                w11v = wv[pl.ds(wbase + 3 * LANES, LANES)]
                for j in range(LANES):
                    p = v * LANES + j
                    w00 = w00v[j]
                    w01 = w01v[j]
                    w10 = w10v[j]
                    w11 = w11v[j]
                    for f in range(FDIM // LANES):
                        sl = pl.ds(f * LANES, LANES)
                        acc = (rows[4 * i + 0, p, sl] * w00
                               + rows[4 * i + 1, p, sl] * w01
                               + rows[4 * i + 2, p, sl] * w10
                               + rows[4 * i + 3, p, sl] * w11)
                        outv[p, pl.ds(i * FDIM + f * LANES, LANES)] = acc

        pltpu.async_copy(outv, out_hbm.at[pl.ds(start, C)], semo)
        return carry

    lax.fori_loop(0, NCHUNK, chunk_body, 0)
    pltpu.make_async_copy(
        outv, out_hbm.at[pl.ds(base + (NCHUNK - 1) * C, C)], semo).wait()


def _run(x, fm):
    xT = x.T
    tab = fm.reshape(3 * ROWS_PER_PLANE, FDIM)
    mesh = plsc.VectorSubcoreMesh(core_axis_name="c", subcore_axis_name="s")
    kfn = pl.kernel(
        _sc_body,
        out_type=jax.ShapeDtypeStruct((N_POINTS, OUT_DIM), jnp.float32),
        mesh=mesh,
        scratch_types=[
            pltpu.VMEM((3, C), jnp.float32),
            pltpu.VMEM((12, C), jnp.int32),
            pltpu.VMEM((3 * (C // LANES) * 4 * LANES,), jnp.float32),
            pltpu.VMEM((12, C, FDIM), jnp.float32),
            pltpu.VMEM((C, OUT_DIM), jnp.float32),
            pltpu.SemaphoreType.DMA((3,)),
            pltpu.SemaphoreType.DMA,
        ],
        compiler_params=pltpu.CompilerParams(use_tc_tiling_on_sc=False),
    )
    return kfn(xT, tab)


def kernel(x, level, fm):
    del level
    return _run(x, fm)
